# Initial kernel scaffold; baseline (speedup 1.0000x reference)
import jax
import jax.numpy as jnp
from jax.experimental import pallas as pl


def kernel(x, batch, W0, b0, W1, b1):
    raise NotImplementedError("write your pallas kernel here")


# speedup vs baseline: 5.9857x; 5.9857x over previous
import jax
import jax.numpy as jnp
from jax import lax
from jax.experimental import pallas as pl
from jax.experimental.pallas import tpu as pltpu

N = 16384
F = 512
H = 128
S = 8
BLK = 1024
NBLK = N // BLK


def _segsum_body(b3_ref, x_ref, out_ref):
    i = pl.program_id(0)
    ids = b3_ref[0, 0, :]
    oh = (ids[:, None] == lax.broadcasted_iota(jnp.int32, (BLK, S), 1)
          ).astype(jnp.float32)
    part = lax.dot_general(oh, x_ref[...], (((0,), (0,)), ((), ())),
                           preferred_element_type=jnp.float32)

    @pl.when(i == 0)
    def _():
        out_ref[...] = part

    @pl.when(i != 0)
    def _():
        out_ref[...] += part


def _apply_body(b2_ref, sums_ref, W0_ref, b0_ref, W1_ref, b1_ref,
                b3_ref, x_ref, out_ref, gate_ref):
    i = pl.program_id(0)

    @pl.when(i == 0)
    def _():
        b2 = b2_ref[...]
        cnt = jnp.concatenate(
            [jnp.sum((b2 == s).astype(jnp.float32))[None] for s in range(S)]
        )
        mean = sums_ref[...] / jnp.maximum(cnt, 1.0)[:, None]
        h = jnp.maximum(
            lax.dot_general(mean, W0_ref[...], (((1,), (0,)), ((), ())),
                            preferred_element_type=jnp.float32)
            + b0_ref[...], 0.0)
        z = lax.dot_general(h, W1_ref[...], (((1,), (0,)), ((), ())),
                            preferred_element_type=jnp.float32) + b1_ref[...]
        gate_ref[...] = 1.0 / (1.0 + jnp.exp(-z))

    ids = b3_ref[0, 0, :]
    oh = (ids[:, None] == lax.broadcasted_iota(jnp.int32, (BLK, S), 1)
          ).astype(jnp.float32)
    y = lax.dot_general(oh, gate_ref[...], (((1,), (0,)), ((), ())),
                        preferred_element_type=jnp.float32)
    out_ref[...] = x_ref[...] * y


def kernel(x, batch, W0, b0, W1, b1):
    batch32 = batch.astype(jnp.int32)
    b3 = batch32.reshape(NBLK, 1, BLK)
    b2 = batch32.reshape(128, 128)

    sums = pl.pallas_call(
        _segsum_body,
        grid=(NBLK,),
        in_specs=[
            pl.BlockSpec((1, 1, BLK), lambda i: (i, 0, 0)),
            pl.BlockSpec((BLK, F), lambda i: (i, 0)),
        ],
        out_specs=pl.BlockSpec((S, F), lambda i: (0, 0)),
        out_shape=jax.ShapeDtypeStruct((S, F), jnp.float32),
    )(b3, x)

    out = pl.pallas_call(
        _apply_body,
        grid=(NBLK,),
        in_specs=[
            pl.BlockSpec((128, 128), lambda i: (0, 0)),
            pl.BlockSpec((S, F), lambda i: (0, 0)),
            pl.BlockSpec((F, H), lambda i: (0, 0)),
            pl.BlockSpec((1, H), lambda i: (0, 0)),
            pl.BlockSpec((H, F), lambda i: (0, 0)),
            pl.BlockSpec((1, F), lambda i: (0, 0)),
            pl.BlockSpec((1, 1, BLK), lambda i: (i, 0, 0)),
            pl.BlockSpec((BLK, F), lambda i: (i, 0)),
        ],
        out_specs=pl.BlockSpec((BLK, F), lambda i: (i, 0)),
        out_shape=jax.ShapeDtypeStruct((N, F), jnp.float32),
        scratch_shapes=[pltpu.VMEM((S, F), jnp.float32)],
    )(b2, sums, W0, b0.reshape(1, H), W1, b1.reshape(1, F), b3, x)

    return out


# speedup vs baseline: 7.2150x; 1.2054x over previous
import jax
import jax.numpy as jnp
from jax import lax
from jax.experimental import pallas as pl
from jax.experimental.pallas import tpu as pltpu

N = 16384
F = 512
H = 128
S = 8
BLK = 1024
NBLK = N // BLK


def _fused_body(b2_ref, x_ref, W0_ref, b0_ref, W1_ref, b1_ref, b3_ref,
                out_ref, acc_ref, gate_ref):
    p = pl.program_id(0)
    i = pl.program_id(1)
    ids = b3_ref[0, 0, :]
    oh = (ids[:, None] == lax.broadcasted_iota(jnp.int32, (BLK, S), 1)
          ).astype(jnp.float32)

    @pl.when(p == 0)
    def _():
        xi = x_ref[pl.ds(i * BLK, BLK), :]
        part = lax.dot_general(oh, xi, (((0,), (0,)), ((), ())),
                               preferred_element_type=jnp.float32)

        @pl.when(i == 0)
        def _():
            acc_ref[...] = part

        @pl.when(i != 0)
        def _():
            acc_ref[...] += part

    @pl.when((p == 1) & (i == 0))
    def _():
        b2 = b2_ref[...]
        cnt = jnp.concatenate(
            [jnp.sum((b2 == s).astype(jnp.float32))[None] for s in range(S)]
        )
        mean = acc_ref[...] / jnp.maximum(cnt, 1.0)[:, None]
        h = jnp.maximum(
            lax.dot_general(mean, W0_ref[...], (((1,), (0,)), ((), ())),
                            preferred_element_type=jnp.float32)
            + b0_ref[...], 0.0)
        z = lax.dot_general(h, W1_ref[...], (((1,), (0,)), ((), ())),
                            preferred_element_type=jnp.float32) + b1_ref[...]
        gate_ref[...] = 1.0 / (1.0 + jnp.exp(-z))

    @pl.when(p == 1)
    def _():
        y = lax.dot_general(oh, gate_ref[...], (((1,), (0,)), ((), ())),
                            preferred_element_type=jnp.float32)
        out_ref[...] = x_ref[pl.ds(i * BLK, BLK), :] * y


def kernel(x, batch, W0, b0, W1, b1):
    batch32 = batch.astype(jnp.int32)
    b3 = batch32.reshape(NBLK, 1, BLK)
    b2 = batch32.reshape(128, 128)

    out = pl.pallas_call(
        _fused_body,
        grid=(2, NBLK),
        in_specs=[
            pl.BlockSpec((128, 128), lambda p, i: (0, 0)),
            pl.BlockSpec((N, F), lambda p, i: (0, 0)),
            pl.BlockSpec((F, H), lambda p, i: (0, 0)),
            pl.BlockSpec((1, H), lambda p, i: (0, 0)),
            pl.BlockSpec((H, F), lambda p, i: (0, 0)),
            pl.BlockSpec((1, F), lambda p, i: (0, 0)),
            pl.BlockSpec((1, 1, BLK), lambda p, i: (i, 0, 0)),
        ],
        out_specs=pl.BlockSpec((BLK, F), lambda p, i: (p * i, 0)),
        out_shape=jax.ShapeDtypeStruct((N, F), jnp.float32),
        scratch_shapes=[pltpu.VMEM((S, F), jnp.float32),
                        pltpu.VMEM((S, F), jnp.float32)],
    )(b2, x, W0, b0.reshape(1, H), W1, b1.reshape(1, F), b3)

    return out


# speedup vs baseline: 7.4289x; 1.0296x over previous
import jax
import jax.numpy as jnp
from jax import lax
from jax.experimental import pallas as pl
from jax.experimental.pallas import tpu as pltpu

N = 16384
F = 512
H = 128
S = 8
BLK = 1024
NBLK = N // BLK


def _fused_body(b2_ref, x_ref, W0_ref, b0_ref, W1_ref, b1_ref, b3_ref,
                out_ref, xsave_ref, acc_ref, gate_ref):
    p = pl.program_id(0)
    i = pl.program_id(1)
    ids = b3_ref[0, 0, :]
    oh = (ids[:, None] == lax.broadcasted_iota(jnp.int32, (BLK, S), 1)
          ).astype(jnp.float32)

    @pl.when(p == 0)
    def _():
        xi = x_ref[...]
        xsave_ref[pl.ds(i * BLK, BLK), :] = xi
        part = lax.dot_general(oh, xi, (((0,), (0,)), ((), ())),
                               preferred_element_type=jnp.float32)

        @pl.when(i == 0)
        def _():
            acc_ref[...] = part

        @pl.when(i != 0)
        def _():
            acc_ref[...] += part

    @pl.when((p == 1) & (i == 0))
    def _():
        b2 = b2_ref[...]
        cnt = jnp.concatenate(
            [jnp.sum((b2 == s).astype(jnp.float32))[None] for s in range(S)]
        )
        mean = acc_ref[...] / jnp.maximum(cnt, 1.0)[:, None]
        h = jnp.maximum(
            lax.dot_general(mean, W0_ref[...], (((1,), (0,)), ((), ())),
                            preferred_element_type=jnp.float32)
            + b0_ref[...], 0.0)
        z = lax.dot_general(h, W1_ref[...], (((1,), (0,)), ((), ())),
                            preferred_element_type=jnp.float32) + b1_ref[...]
        gate_ref[...] = 1.0 / (1.0 + jnp.exp(-z))

    @pl.when(p == 1)
    def _():
        y = lax.dot_general(oh, gate_ref[...], (((1,), (0,)), ((), ())),
                            preferred_element_type=jnp.float32)
        out_ref[...] = xsave_ref[pl.ds(i * BLK, BLK), :] * y


def kernel(x, batch, W0, b0, W1, b1):
    batch32 = batch.astype(jnp.int32)
    b3 = batch32.reshape(NBLK, 1, BLK)
    b2 = batch32.reshape(128, 128)

    out = pl.pallas_call(
        _fused_body,
        grid=(2, NBLK),
        in_specs=[
            pl.BlockSpec((128, 128), lambda p, i: (0, 0)),
            pl.BlockSpec((BLK, F), lambda p, i: (i * (1 - p) + (NBLK - 1) * p, 0)),
            pl.BlockSpec((F, H), lambda p, i: (0, 0)),
            pl.BlockSpec((1, H), lambda p, i: (0, 0)),
            pl.BlockSpec((H, F), lambda p, i: (0, 0)),
            pl.BlockSpec((1, F), lambda p, i: (0, 0)),
            pl.BlockSpec((1, 1, BLK), lambda p, i: (i, 0, 0)),
        ],
        out_specs=pl.BlockSpec((BLK, F), lambda p, i: (p * i, 0)),
        out_shape=jax.ShapeDtypeStruct((N, F), jnp.float32),
        scratch_shapes=[pltpu.VMEM((N, F), jnp.float32),
                        pltpu.VMEM((S, F), jnp.float32),
                        pltpu.VMEM((S, F), jnp.float32)],
    )(b2, x, W0, b0.reshape(1, H), W1, b1.reshape(1, F), b3)

    return out


# speedup vs baseline: 10.9311x; 1.4714x over previous
import jax
import jax.numpy as jnp
from jax import lax
from jax.experimental import pallas as pl
from jax.experimental.pallas import tpu as pltpu

N = 16384
F = 512
H = 128
S = 8
BLK = 1024
NBLK = N // BLK


def _body(b2_ref, b3_ref, W0_ref, b0_ref, W1_ref, b1_ref, x_hbm, out_hbm,
          xbuf_ref, rsem, wsem):
    read_copies = []
    for k in range(NBLK):
        c = pltpu.make_async_copy(
            x_hbm.at[pl.ds(k * BLK, BLK), :],
            xbuf_ref.at[pl.ds(k * BLK, BLK), :],
            rsem.at[k])
        c.start()
        read_copies.append(c)

    def onehot(k):
        ids = b3_ref[k, 0, :]
        return (ids[:, None] == lax.broadcasted_iota(jnp.int32, (BLK, S), 1)
                ).astype(jnp.float32)

    acc = jnp.zeros((S, F), jnp.float32)
    for k in range(NBLK):
        read_copies[k].wait()
        xi = xbuf_ref[pl.ds(k * BLK, BLK), :]
        acc = acc + lax.dot_general(onehot(k), xi, (((0,), (0,)), ((), ())),
                                    preferred_element_type=jnp.float32)

    b2 = b2_ref[...]
    cnt = jnp.concatenate(
        [jnp.sum((b2 == s).astype(jnp.float32))[None] for s in range(S)])
    mean = acc / jnp.maximum(cnt, 1.0)[:, None]
    h = jnp.maximum(
        lax.dot_general(mean, W0_ref[...], (((1,), (0,)), ((), ())),
                        preferred_element_type=jnp.float32) + b0_ref[...],
        0.0)
    z = lax.dot_general(h, W1_ref[...], (((1,), (0,)), ((), ())),
                        preferred_element_type=jnp.float32) + b1_ref[...]
    gate = 1.0 / (1.0 + jnp.exp(-z))

    write_copies = []
    for k in range(NBLK):
        y = lax.dot_general(onehot(k), gate, (((1,), (0,)), ((), ())),
                            preferred_element_type=jnp.float32)
        xbuf_ref[pl.ds(k * BLK, BLK), :] *= y
        c = pltpu.make_async_copy(
            xbuf_ref.at[pl.ds(k * BLK, BLK), :],
            out_hbm.at[pl.ds(k * BLK, BLK), :],
            wsem.at[k])
        c.start()
        write_copies.append(c)

    for c in write_copies:
        c.wait()


def kernel(x, batch, W0, b0, W1, b1):
    batch32 = batch.astype(jnp.int32)
    b3 = batch32.reshape(NBLK, 1, BLK)
    b2 = batch32.reshape(128, 128)

    out = pl.pallas_call(
        _body,
        in_specs=[
            pl.BlockSpec(memory_space=pltpu.MemorySpace.VMEM),
            pl.BlockSpec(memory_space=pltpu.MemorySpace.VMEM),
            pl.BlockSpec(memory_space=pltpu.MemorySpace.VMEM),
            pl.BlockSpec(memory_space=pltpu.MemorySpace.VMEM),
            pl.BlockSpec(memory_space=pltpu.MemorySpace.VMEM),
            pl.BlockSpec(memory_space=pltpu.MemorySpace.VMEM),
            pl.BlockSpec(memory_space=pltpu.MemorySpace.HBM),
        ],
        out_specs=pl.BlockSpec(memory_space=pltpu.MemorySpace.HBM),
        out_shape=jax.ShapeDtypeStruct((N, F), jnp.float32),
        scratch_shapes=[
            pltpu.VMEM((N, F), jnp.float32),
            pltpu.SemaphoreType.DMA((NBLK,)),
            pltpu.SemaphoreType.DMA((NBLK,)),
        ],
    )(b2, b3, W0, b0.reshape(1, H), W1, b1.reshape(1, F), x)

    return out
